# baseline (speedup 1.0000x reference)
import functools

import jax
import jax.numpy as jnp
from jax import lax
from jax.experimental import pallas as pl
from jax.experimental.pallas import tpu as pltpu
from jax.experimental.pallas import tpu_sc as plsc

N_NODES = 10000
N_EDGES = 320000
D = 128
MAX_Z = 16

_NUM_CORES = 2
_NUM_SUBCORES = 16
_NUM_TILES = _NUM_CORES * _NUM_SUBCORES
_CHUNK = 128
_NC0 = 158
_NC1 = 2
_PT0 = _NC0 * _CHUNK
_PT1 = _NC1 * _CHUNK
_E_PAD = (_PT0 + _PT1) * _NUM_SUBCORES
_N_PAD = 10240
_STRIPE = _N_PAD // _NUM_SUBCORES


def _nrm(x):
    return jnp.clip(jnp.sqrt(jnp.sum(x * x, axis=-1, keepdims=True)), 1e-15, None)


def _artanh(x):
    x = jnp.clip(x, -1.0 + 1e-7, 1.0 - 1e-7)
    return 0.5 * (jnp.log1p(x) - jnp.log1p(-x))


def _proj(x):
    n = _nrm(x)
    maxnorm = 1.0 - 1e-5
    return jnp.where(n > maxnorm, x / n * maxnorm, x)


def _expmap0(u):
    n = _nrm(u)
    return _proj(jnp.tanh(n) * u / n)


def _logmap0(p):
    n = _nrm(p)
    return _artanh(n) * p / n


def _mobius_add(x, y):
    x2 = jnp.sum(x * x, axis=-1, keepdims=True)
    y2 = jnp.sum(y * y, axis=-1, keepdims=True)
    xy = jnp.sum(x * y, axis=-1, keepdims=True)
    num = (1 + 2 * xy + y2) * x + (1 - x2) * y
    denom = 1 + 2 * xy + x2 * y2
    return num / jnp.clip(denom, 1e-15, None)


def _stage_a_body(h_ref, wt_ref, b_ref, o_ref):
    x = _proj(_expmap0(h_ref[...]))
    u = _logmap0(x)
    mv = _expmap0(jnp.dot(u, wt_ref[...], preferred_element_type=jnp.float32))
    hb = _expmap0(b_ref[...])
    h1 = _proj(_mobius_add(mv, hb))
    o_ref[...] = _logmap0(h1)


def _stage_b_body(p0_ref, p1_ref, nm_ref, wt_ref, b_ref, o_ref):
    agg = (p0_ref[...] + p1_ref[...]) * nm_ref[...]
    h2 = _proj(_expmap0(agg))
    h3 = _proj(_expmap0(jax.nn.relu(_logmap0(h2))))
    u = _logmap0(h3)
    mv = _expmap0(jnp.dot(u, wt_ref[...], preferred_element_type=jnp.float32))
    hb = _expmap0(b_ref[...])
    h1 = _proj(_mobius_add(mv, hb))
    o_ref[...] = _logmap0(h1)


def _stage_c_body(p0_ref, p1_ref, nm_ref, wt_ref, b_ref, o_ref):
    agg = (p0_ref[...] + p1_ref[...]) * nm_ref[...]
    h2 = _proj(_expmap0(agg))
    h3 = _proj(_expmap0(jax.nn.relu(_logmap0(h2))))
    out_t = _logmap0(h3)
    o_ref[...] = (
        jnp.dot(out_t, wt_ref[...], preferred_element_type=jnp.float32)
        + b_ref[...]
    )


def _stage_a(h, w1t, b1):
    return pl.pallas_call(
        _stage_a_body,
        out_shape=jax.ShapeDtypeStruct((N_NODES, D), jnp.float32),
    )(h, w1t, b1)


def _stage_b(p0, p1, nm, w2t, b2):
    return pl.pallas_call(
        _stage_b_body,
        out_shape=jax.ShapeDtypeStruct((N_NODES, D), jnp.float32),
    )(p0, p1, nm, w2t, b2)


def _stage_c(p0, p1, nm, wot, bo):
    return pl.pallas_call(
        _stage_c_body,
        out_shape=jax.ShapeDtypeStruct((N_NODES, MAX_Z), jnp.float32),
    )(p0, p1, nm, wot, bo)


def _sc_seg_body(xt_hbm, src_hbm, dst_hbm, w_hbm, out_hbm, agg_sh,
                 s0, s1, d0, d1, w0, w1, r0b, r1b,
                 ss0, ss1, ds0, ds1, ws0, ws1, gs0, gs1):
    cid = lax.axis_index("c")
    sid = lax.axis_index("s")
    sbufs = (s0, s1)
    dbufs = (d0, d1)
    wbufs = (w0, w1)
    rbufs = (r0b, r1b)
    ssems = (ss0, ss1)
    dsems = (ds0, ds1)
    wsems = (ws0, ws1)
    gsems = (gs0, gs1)

    def _zrow(r, carry):
        for j in range(D // 16):
            r0b[r, pl.ds(j * 16, 16)] = jnp.zeros((16,), jnp.float32)
        return carry

    lax.fori_loop(0, _CHUNK, _zrow, 0)
    for k in range(_STRIPE // _CHUNK):
        rr = sid * _STRIPE + k * _CHUNK
        pltpu.sync_copy(r0b, agg_sh.at[pl.ds(rr, _CHUNK)])
    plsc.subcore_barrier()

    pt = jnp.where(cid == 0, _PT0, _PT1)
    nchunk = jnp.where(cid == 0, _NC0, _NC1)
    base0 = cid * _NUM_SUBCORES * _PT0 + sid * pt

    def _scopy_start(ci, b):
        base = base0 + ci * _CHUNK
        pltpu.async_copy(src_hbm.at[pl.ds(base, _CHUNK)], sbufs[b], ssems[b])

    def _scopy_wait(b):
        pltpu.make_async_copy(src_hbm.at[pl.ds(0, _CHUNK)], sbufs[b],
                              ssems[b]).wait()

    def _dwcopy_start(ci, b):
        base = base0 + ci * _CHUNK
        pltpu.async_copy(dst_hbm.at[pl.ds(base, _CHUNK)], dbufs[b], dsems[b])
        pltpu.async_copy(w_hbm.at[pl.ds(base, _CHUNK)], wbufs[b], wsems[b])

    def _dwcopy_wait(b):
        pltpu.make_async_copy(dst_hbm.at[pl.ds(0, _CHUNK)], dbufs[b],
                              dsems[b]).wait()
        pltpu.make_async_copy(w_hbm.at[pl.ds(0, _CHUNK)], wbufs[b],
                              wsems[b]).wait()

    def _gather_start(b):
        pltpu.async_copy(xt_hbm.at[sbufs[b]], rbufs[b], gsems[b])

    def _gather_wait(b):
        pltpu.make_async_copy(xt_hbm.at[sbufs[b]], rbufs[b],
                              gsems[b]).wait()

    def _scatter(b):
        pltpu.sync_copy(rbufs[b], agg_sh.at[dbufs[b]], add=True)

    def _scale(b):
        rows = rbufs[b]
        wvb = wbufs[b]

        def _srow(t, c2):
            for u in range(2):
                i = t * 2 + u
                g = i // 16
                lane = i - g * 16
                wv = wvb[pl.ds(g * 16, 16)]
                ws = wv.at[lane + jnp.zeros((16,), jnp.int32)].get(
                    mode="promise_in_bounds")
                for j in range(D // 16):
                    sl = pl.ds(j * 16, 16)
                    rows[i, sl] = rows[i, sl] * ws
            return c2

        lax.fori_loop(0, _CHUNK // 2, _srow, 0)

    _scopy_start(0, 0)
    _scopy_wait(0)
    _dwcopy_start(0, 0)
    _gather_start(0)
    _scopy_start(1, 1)

    def _pair(t, carry):
        k0 = t * 2
        for b in (0, 1):
            k = k0 + b
            o = 1 - b

            @pl.when(k + 1 < nchunk)
            def _():
                _dwcopy_start(k + 1, o)
                _scopy_wait(o)
                _gather_start(o)

            _gather_wait(b)

            @pl.when(k + 2 < nchunk)
            def _():
                _scopy_start(k + 2, b)

            _dwcopy_wait(b)
            _scale(b)
            _scatter(b)
        return carry

    lax.fori_loop(0, nchunk // 2, _pair, 0)
    plsc.subcore_barrier()

    for k in range(_STRIPE // _CHUNK):
        rr = sid * _STRIPE + k * _CHUNK
        pltpu.sync_copy(agg_sh.at[pl.ds(rr, _CHUNK)],
                        out_hbm.at[cid, pl.ds(rr, _CHUNK)])


@functools.partial(
    pl.kernel,
    out_type=jax.ShapeDtypeStruct((_NUM_CORES, _N_PAD, D), jnp.float32),
    mesh=plsc.VectorSubcoreMesh(core_axis_name="c", subcore_axis_name="s"),
    scratch_types=[
        pltpu.VMEM_SHARED((_N_PAD, D), jnp.float32),
        pltpu.VMEM((_CHUNK,), jnp.int32),
        pltpu.VMEM((_CHUNK,), jnp.int32),
        pltpu.VMEM((_CHUNK,), jnp.int32),
        pltpu.VMEM((_CHUNK,), jnp.int32),
        pltpu.VMEM((_CHUNK,), jnp.float32),
        pltpu.VMEM((_CHUNK,), jnp.float32),
        pltpu.VMEM((_CHUNK, D), jnp.float32),
        pltpu.VMEM((_CHUNK, D), jnp.float32),
        pltpu.SemaphoreType.DMA,
        pltpu.SemaphoreType.DMA,
        pltpu.SemaphoreType.DMA,
        pltpu.SemaphoreType.DMA,
        pltpu.SemaphoreType.DMA,
        pltpu.SemaphoreType.DMA,
        pltpu.SemaphoreType.DMA,
        pltpu.SemaphoreType.DMA,
    ],
)
def _sc_seg(xt_hbm, src_hbm, dst_hbm, w_hbm, out_hbm, agg_sh,
            s0, s1, d0, d1, w0, w1, r0b, r1b,
            ss0, ss1, ds0, ds1, ws0, ws1, gs0, gs1):
    _sc_seg_body(xt_hbm, src_hbm, dst_hbm, w_hbm, out_hbm, agg_sh,
                 s0, s1, d0, d1, w0, w1, r0b, r1b,
                 ss0, ss1, ds0, ds1, ws0, ws1, gs0, gs1)


def _stage_w_body(d_ref, m_ref, o_ref):
    o_ref[...] = jnp.exp(-d_ref[...]) * m_ref[...]


def _stage_w(dist_p, em_p):
    d2 = dist_p.reshape(_E_PAD // D, D)
    m2 = em_p.reshape(_E_PAD // D, D)
    w2 = pl.pallas_call(
        _stage_w_body,
        out_shape=jax.ShapeDtypeStruct((_E_PAD // D, D), jnp.float32),
    )(d2, m2)
    return w2.reshape(_E_PAD)


def kernel(h, distances, edges, node_mask, edge_mask,
           W1, b1, W2, b2, W_out, b_out):
    src = edges[0].astype(jnp.int32)
    dst = edges[1].astype(jnp.int32)
    dist = distances.reshape(-1).astype(jnp.float32)
    em = edge_mask.reshape(-1).astype(jnp.float32)
    pad = _E_PAD - N_EDGES
    src_p = jnp.pad(src, (0, pad))
    dst_p = jnp.pad(dst, (0, pad))
    dist_p = jnp.pad(dist, (0, pad))
    em_p = jnp.pad(em, (0, pad))
    w_p = _stage_w(dist_p, em_p)

    w1t = W1.T
    w2t = W2.T
    wot = W_out.T
    b1r = b1.reshape(1, D)
    b2r = b2.reshape(1, D)
    bor = b_out.reshape(1, MAX_Z)
    nm = node_mask.astype(jnp.float32)

    xt1 = _stage_a(h, w1t, b1r)
    parts1 = _sc_seg(xt1, src_p, dst_p, w_p)
    xt2 = _stage_b(parts1[0, :N_NODES], parts1[1, :N_NODES], nm, w2t, b2r)
    parts2 = _sc_seg(xt2, src_p, dst_p, w_p)
    return _stage_c(parts2[0, :N_NODES], parts2[1, :N_NODES], nm, wot, bor)


# speedup vs baseline: 1.2137x; 1.2137x over previous
import functools

import jax
import jax.numpy as jnp
from jax import lax
from jax.experimental import pallas as pl
from jax.experimental.pallas import tpu as pltpu
from jax.experimental.pallas import tpu_sc as plsc

N_NODES = 10000
N_EDGES = 320000
D = 128
MAX_Z = 16

_NUM_CORES = 2
_NUM_SUBCORES = 16
_NUM_TILES = _NUM_CORES * _NUM_SUBCORES
_CHUNK = 128
_NC0 = 130
_NC1 = 30
_PT0 = _NC0 * _CHUNK
_PT1 = _NC1 * _CHUNK
_E_PAD = (_PT0 + _PT1) * _NUM_SUBCORES
_N_PAD = 10240
_STRIPE = _N_PAD // _NUM_SUBCORES


def _nrm(x):
    return jnp.clip(jnp.sqrt(jnp.sum(x * x, axis=-1, keepdims=True)), 1e-15, None)


def _artanh(x):
    x = jnp.clip(x, -1.0 + 1e-7, 1.0 - 1e-7)
    return 0.5 * (jnp.log1p(x) - jnp.log1p(-x))


def _proj(x):
    n = _nrm(x)
    maxnorm = 1.0 - 1e-5
    return jnp.where(n > maxnorm, x / n * maxnorm, x)


def _expmap0(u):
    n = _nrm(u)
    return _proj(jnp.tanh(n) * u / n)


def _logmap0(p):
    n = _nrm(p)
    return _artanh(n) * p / n


def _mobius_add(x, y):
    x2 = jnp.sum(x * x, axis=-1, keepdims=True)
    y2 = jnp.sum(y * y, axis=-1, keepdims=True)
    xy = jnp.sum(x * y, axis=-1, keepdims=True)
    num = (1 + 2 * xy + y2) * x + (1 - x2) * y
    denom = 1 + 2 * xy + x2 * y2
    return num / jnp.clip(denom, 1e-15, None)


def _stage_a_body(h_ref, wt_ref, b_ref, o_ref):
    x = _proj(_expmap0(h_ref[...]))
    u = _logmap0(x)
    mv = _expmap0(jnp.dot(u, wt_ref[...], preferred_element_type=jnp.float32))
    hb = _expmap0(b_ref[...])
    h1 = _proj(_mobius_add(mv, hb))
    o_ref[...] = _logmap0(h1)


def _stage_b_body(p0_ref, p1_ref, nm_ref, wt_ref, b_ref, o_ref):
    agg = (p0_ref[...] + p1_ref[...]) * nm_ref[...]
    h2 = _proj(_expmap0(agg))
    h3 = _proj(_expmap0(jax.nn.relu(_logmap0(h2))))
    u = _logmap0(h3)
    mv = _expmap0(jnp.dot(u, wt_ref[...], preferred_element_type=jnp.float32))
    hb = _expmap0(b_ref[...])
    h1 = _proj(_mobius_add(mv, hb))
    o_ref[...] = _logmap0(h1)


def _stage_c_body(p0_ref, p1_ref, nm_ref, wt_ref, b_ref, o_ref):
    agg = (p0_ref[...] + p1_ref[...]) * nm_ref[...]
    h2 = _proj(_expmap0(agg))
    h3 = _proj(_expmap0(jax.nn.relu(_logmap0(h2))))
    out_t = _logmap0(h3)
    o_ref[...] = (
        jnp.dot(out_t, wt_ref[...], preferred_element_type=jnp.float32)
        + b_ref[...]
    )


def _stage_a(h, w1t, b1):
    return pl.pallas_call(
        _stage_a_body,
        out_shape=jax.ShapeDtypeStruct((N_NODES, D), jnp.float32),
    )(h, w1t, b1)


def _stage_b(p0, p1, nm, w2t, b2):
    return pl.pallas_call(
        _stage_b_body,
        out_shape=jax.ShapeDtypeStruct((N_NODES, D), jnp.float32),
    )(p0, p1, nm, w2t, b2)


def _stage_c(p0, p1, nm, wot, bo):
    return pl.pallas_call(
        _stage_c_body,
        out_shape=jax.ShapeDtypeStruct((N_NODES, MAX_Z), jnp.float32),
    )(p0, p1, nm, wot, bo)


def _sc_seg_body(xt_hbm, src_hbm, dst_hbm, w_hbm, out_hbm, agg_sh,
                 s0, s1, d0, d1, w0, w1, r0b, r1b,
                 ss0, ss1, ds0, ds1, ws0, ws1, gs0, gs1):
    cid = lax.axis_index("c")
    sid = lax.axis_index("s")
    sbufs = (s0, s1)
    dbufs = (d0, d1)
    wbufs = (w0, w1)
    rbufs = (r0b, r1b)
    ssems = (ss0, ss1)
    dsems = (ds0, ds1)
    wsems = (ws0, ws1)
    gsems = (gs0, gs1)

    def _zrow(r, carry):
        for j in range(D // 16):
            r0b[r, pl.ds(j * 16, 16)] = jnp.zeros((16,), jnp.float32)
        return carry

    lax.fori_loop(0, _CHUNK, _zrow, 0)
    for k in range(_STRIPE // _CHUNK):
        rr = sid * _STRIPE + k * _CHUNK
        pltpu.sync_copy(r0b, agg_sh.at[pl.ds(rr, _CHUNK)])
    plsc.subcore_barrier()

    pt = jnp.where(cid == 0, _PT0, _PT1)
    nchunk = jnp.where(cid == 0, _NC0, _NC1)
    base0 = cid * _NUM_SUBCORES * _PT0 + sid * pt

    def _scopy_start(ci, b):
        base = base0 + ci * _CHUNK
        pltpu.async_copy(src_hbm.at[pl.ds(base, _CHUNK)], sbufs[b], ssems[b])

    def _scopy_wait(b):
        pltpu.make_async_copy(src_hbm.at[pl.ds(0, _CHUNK)], sbufs[b],
                              ssems[b]).wait()

    def _dwcopy_start(ci, b):
        base = base0 + ci * _CHUNK
        pltpu.async_copy(dst_hbm.at[pl.ds(base, _CHUNK)], dbufs[b], dsems[b])
        pltpu.async_copy(w_hbm.at[pl.ds(base, _CHUNK)], wbufs[b], wsems[b])

    def _dwcopy_wait(b):
        pltpu.make_async_copy(dst_hbm.at[pl.ds(0, _CHUNK)], dbufs[b],
                              dsems[b]).wait()
        pltpu.make_async_copy(w_hbm.at[pl.ds(0, _CHUNK)], wbufs[b],
                              wsems[b]).wait()

    def _gather_start(b):
        pltpu.async_copy(xt_hbm.at[sbufs[b]], rbufs[b], gsems[b])

    def _gather_wait(b):
        pltpu.make_async_copy(xt_hbm.at[sbufs[b]], rbufs[b],
                              gsems[b]).wait()

    def _scatter(b):
        pltpu.sync_copy(rbufs[b], agg_sh.at[dbufs[b]], add=True)

    def _scale(b):
        rows = rbufs[b]
        wvb = wbufs[b]

        def _srow(t, c2):
            for u in range(2):
                i = t * 2 + u
                g = i // 16
                lane = i - g * 16
                wv = wvb[pl.ds(g * 16, 16)]
                ws = wv.at[lane + jnp.zeros((16,), jnp.int32)].get(
                    mode="promise_in_bounds")
                for j in range(D // 16):
                    sl = pl.ds(j * 16, 16)
                    rows[i, sl] = rows[i, sl] * ws
            return c2

        lax.fori_loop(0, _CHUNK // 2, _srow, 0)

    _scopy_start(0, 0)
    _scopy_wait(0)
    _dwcopy_start(0, 0)
    _gather_start(0)
    _scopy_start(1, 1)

    def _pair(t, carry):
        k0 = t * 2
        for b in (0, 1):
            k = k0 + b
            o = 1 - b

            @pl.when(k + 1 < nchunk)
            def _():
                _dwcopy_start(k + 1, o)
                _scopy_wait(o)
                _gather_start(o)

            _gather_wait(b)

            @pl.when(k + 2 < nchunk)
            def _():
                _scopy_start(k + 2, b)

            _dwcopy_wait(b)
            _scale(b)
            _scatter(b)
        return carry

    lax.fori_loop(0, nchunk // 2, _pair, 0)
    plsc.subcore_barrier()

    for k in range(_STRIPE // _CHUNK):
        rr = sid * _STRIPE + k * _CHUNK
        pltpu.sync_copy(agg_sh.at[pl.ds(rr, _CHUNK)],
                        out_hbm.at[cid, pl.ds(rr, _CHUNK)])


@functools.partial(
    pl.kernel,
    out_type=jax.ShapeDtypeStruct((_NUM_CORES, _N_PAD, D), jnp.float32),
    mesh=plsc.VectorSubcoreMesh(core_axis_name="c", subcore_axis_name="s"),
    scratch_types=[
        pltpu.VMEM_SHARED((_N_PAD, D), jnp.float32),
        pltpu.VMEM((_CHUNK,), jnp.int32),
        pltpu.VMEM((_CHUNK,), jnp.int32),
        pltpu.VMEM((_CHUNK,), jnp.int32),
        pltpu.VMEM((_CHUNK,), jnp.int32),
        pltpu.VMEM((_CHUNK,), jnp.float32),
        pltpu.VMEM((_CHUNK,), jnp.float32),
        pltpu.VMEM((_CHUNK, D), jnp.float32),
        pltpu.VMEM((_CHUNK, D), jnp.float32),
        pltpu.SemaphoreType.DMA,
        pltpu.SemaphoreType.DMA,
        pltpu.SemaphoreType.DMA,
        pltpu.SemaphoreType.DMA,
        pltpu.SemaphoreType.DMA,
        pltpu.SemaphoreType.DMA,
        pltpu.SemaphoreType.DMA,
        pltpu.SemaphoreType.DMA,
    ],
)
def _sc_seg(xt_hbm, src_hbm, dst_hbm, w_hbm, out_hbm, agg_sh,
            s0, s1, d0, d1, w0, w1, r0b, r1b,
            ss0, ss1, ds0, ds1, ws0, ws1, gs0, gs1):
    _sc_seg_body(xt_hbm, src_hbm, dst_hbm, w_hbm, out_hbm, agg_sh,
                 s0, s1, d0, d1, w0, w1, r0b, r1b,
                 ss0, ss1, ds0, ds1, ws0, ws1, gs0, gs1)


def _stage_w_body(d_ref, m_ref, o_ref):
    o_ref[...] = jnp.exp(-d_ref[...]) * m_ref[...]


def _stage_w(dist_p, em_p):
    d2 = dist_p.reshape(_E_PAD // D, D)
    m2 = em_p.reshape(_E_PAD // D, D)
    w2 = pl.pallas_call(
        _stage_w_body,
        out_shape=jax.ShapeDtypeStruct((_E_PAD // D, D), jnp.float32),
    )(d2, m2)
    return w2.reshape(_E_PAD)


def kernel(h, distances, edges, node_mask, edge_mask,
           W1, b1, W2, b2, W_out, b_out):
    src = edges[0].astype(jnp.int32)
    dst = edges[1].astype(jnp.int32)
    dist = distances.reshape(-1).astype(jnp.float32)
    em = edge_mask.reshape(-1).astype(jnp.float32)
    pad = _E_PAD - N_EDGES
    src_p = jnp.pad(src, (0, pad))
    dst_p = jnp.pad(dst, (0, pad))
    dist_p = jnp.pad(dist, (0, pad))
    em_p = jnp.pad(em, (0, pad))
    w_p = _stage_w(dist_p, em_p)

    w1t = W1.T
    w2t = W2.T
    wot = W_out.T
    b1r = b1.reshape(1, D)
    b2r = b2.reshape(1, D)
    bor = b_out.reshape(1, MAX_Z)
    nm = node_mask.astype(jnp.float32)

    xt1 = _stage_a(h, w1t, b1r)
    parts1 = _sc_seg(xt1, src_p, dst_p, w_p)
    xt2 = _stage_b(parts1[0, :N_NODES], parts1[1, :N_NODES], nm, w2t, b2r)
    parts2 = _sc_seg(xt2, src_p, dst_p, w_p)
    return _stage_c(parts2[0, :N_NODES], parts2[1, :N_NODES], nm, wot, bor)
